# baseline (speedup 1.0000x reference)
import functools

import jax
import jax.numpy as jnp
from jax import lax
from jax.experimental import pallas as pl
from jax.experimental.pallas import tpu as pltpu
from jax.experimental.pallas import tpu_sc as plsc

_NC = 2
_NS = 16
_NW = _NC * _NS
_L = 16
_D = 32
_R = 100
_T = 60000
_G = 128
_NG = 15
_PER_W = _G * _NG
_TPAD = _NW * _PER_W


def _body(hidx_hbm, ridx_hbm, tidx_hbm, node_hbm, rel_hbm, out_hbm,
          hidx_v, ridx_v, tidx_v, rel_v, hrows_v, trows_v, score_v, sem):
  wid = lax.axis_index("s") * _NC + lax.axis_index("c")
  base = wid * _PER_W

  pltpu.sync_copy(hidx_hbm.at[pl.ds(base, _PER_W)], hidx_v)
  pltpu.sync_copy(ridx_hbm.at[pl.ds(base, _PER_W)], ridx_v)
  pltpu.sync_copy(tidx_hbm.at[pl.ds(base, _PER_W)], tidx_v)
  pltpu.sync_copy(rel_hbm, rel_v)

  def group(j, carry):
    pltpu.async_copy(node_hbm.at[hidx_v.at[pl.ds(j * _G, _G)]], hrows_v,
                     sem).wait()
    pltpu.async_copy(node_hbm.at[tidx_v.at[pl.ds(j * _G, _G)]], trows_v,
                     sem).wait()

    def step(k, c):
      rows = lax.iota(jnp.int32, _L) + k * _L
      r_ids = ridx_v[pl.ds(j * _G + k * _L, _L)]
      acc = jnp.zeros((_L,), jnp.float32)
      for d in range(_D):
        col = jnp.full((_L,), d, jnp.int32)
        hv = plsc.load_gather(hrows_v, [rows, col])
        tv = plsc.load_gather(trows_v, [rows, col])
        rv = plsc.load_gather(rel_v, [r_ids, col])
        acc = acc + hv * rv * tv
      score_v[pl.ds(j * _G + k * _L, _L)] = acc
      return c

    lax.fori_loop(0, _G // _L, step, 0)
    return carry

  lax.fori_loop(0, _NG, group, 0)
  pltpu.sync_copy(score_v, out_hbm.at[pl.ds(wid * _PER_W, _PER_W)])


@functools.partial(
    pl.kernel,
    out_type=jax.ShapeDtypeStruct((_TPAD,), jnp.float32),
    mesh=plsc.VectorSubcoreMesh(core_axis_name="c", subcore_axis_name="s"),
    compiler_params=pltpu.CompilerParams(
        needs_layout_passes=False, use_tc_tiling_on_sc=False),
    scratch_types=[
        pltpu.VMEM((_PER_W,), jnp.int32),
        pltpu.VMEM((_PER_W,), jnp.int32),
        pltpu.VMEM((_PER_W,), jnp.int32),
        pltpu.VMEM((_R, _D), jnp.float32),
        pltpu.VMEM((_G, _D), jnp.float32),
        pltpu.VMEM((_G, _D), jnp.float32),
        pltpu.VMEM((_PER_W,), jnp.float32),
        pltpu.SemaphoreType.DMA,
    ],
)
def _score_kernel(hidx, ridx, tidx, node, rel, out, *scratch):
  _body(hidx, ridx, tidx, node, rel, out, *scratch)


def kernel(h_idx, r_idx, t_idx, node_embedding, relational_embedding):
  pad = _TPAD - _T
  zpad = jnp.zeros((pad,), jnp.int32)
  h2 = jnp.concatenate([h_idx, zpad])
  r2 = jnp.concatenate([r_idx, zpad])
  t2 = jnp.concatenate([t_idx, zpad])
  score = _score_kernel(h2, r2, t2, node_embedding, relational_embedding)
  return score[:_T]
